# baseline (speedup 1.0000x reference)
import functools

import jax
import jax.numpy as jnp
from jax import lax
from jax.experimental import pallas as pl
from jax.experimental.pallas import tpu as pltpu
from jax.experimental.pallas import tpu_sc as plsc

_N = 10000
_E = 320000
_DIN = 128
_DHID = 128
_NCLS = 64

_NC = 2
_NS = 16
_W = _NC * _NS
_B = 128
_NCH = 80
_EPAD = _W * _B * _NCH
_NPAD = 10112
_RPT = _NPAD // _NS
_SLICES = [(t * _B, min(_B, _RPT - t * _B)) for t in range(-(-_RPT // _B))]

_ROWS = 1000
_G = _N // _ROWS


def _sc_mesh():
    return plsc.VectorSubcoreMesh(core_axis_name="c", subcore_axis_name="s")


@functools.partial(
    pl.kernel,
    mesh=_sc_mesh(),
    out_type=jax.ShapeDtypeStruct((_NC, _NPAD, _DHID), jnp.float32),
    scratch_types=[
        pltpu.VMEM((_NCH, _B), jnp.int32),
        pltpu.VMEM((_B, _DHID), jnp.float32),
        pltpu.VMEM_SHARED((_NPAD, _DHID), jnp.float32),
    ],
)
def _deg_sc(dst_hbm, zeros_hbm, ones_hbm, out_hbm, idx_v, buf_v, deg_sh):
    cid = lax.axis_index("c")
    sid = lax.axis_index("s")
    wid = cid * _NS + sid
    base = sid * _RPT
    pltpu.sync_copy(dst_hbm.at[wid], idx_v)
    pltpu.sync_copy(zeros_hbm, buf_v)
    for off, cnt in _SLICES:
        pltpu.sync_copy(buf_v.at[pl.ds(0, cnt)],
                        deg_sh.at[pl.ds(base + off, cnt)])
    pltpu.sync_copy(ones_hbm, buf_v)
    plsc.subcore_barrier()

    def body(j, carry):
        pltpu.sync_copy(buf_v, deg_sh.at[idx_v.at[j]], add=True)
        return carry

    lax.fori_loop(0, _NCH, body, 0)
    plsc.subcore_barrier()
    for off, cnt in _SLICES:
        pltpu.sync_copy(deg_sh.at[pl.ds(base + off, cnt)],
                        buf_v.at[pl.ds(0, cnt)])
        pltpu.sync_copy(buf_v.at[pl.ds(0, cnt)],
                        out_hbm.at[cid, pl.ds(base + off, cnt)])


def _make_agg_sc(d):
    @functools.partial(
        pl.kernel,
        mesh=_sc_mesh(),
        out_type=jax.ShapeDtypeStruct((_NC, _NPAD, d), jnp.float32),
        scratch_types=[
            pltpu.VMEM((_NCH, _B), jnp.int32),
            pltpu.VMEM((_NCH, _B), jnp.int32),
            pltpu.VMEM((_B, d), jnp.float32),
            pltpu.VMEM_SHARED((_NPAD, d), jnp.float32),
            pltpu.SemaphoreType.DMA,
        ],
    )
    def agg_sc(hs_hbm, src_hbm, dst_hbm, zeros_hbm, out_hbm,
               src_v, dst_v, rows_v, agg_sh, sem):
        cid = lax.axis_index("c")
        sid = lax.axis_index("s")
        wid = cid * _NS + sid
        base = sid * _RPT
        pltpu.sync_copy(src_hbm.at[wid], src_v)
        pltpu.sync_copy(dst_hbm.at[wid], dst_v)
        pltpu.sync_copy(zeros_hbm, rows_v)
        for off, cnt in _SLICES:
            pltpu.sync_copy(rows_v.at[pl.ds(0, cnt)],
                            agg_sh.at[pl.ds(base + off, cnt)])
        plsc.subcore_barrier()

        def body(j, carry):
            pltpu.async_copy(hs_hbm.at[src_v.at[j]], rows_v, sem).wait()
            pltpu.sync_copy(rows_v, agg_sh.at[dst_v.at[j]], add=True)
            return carry

        lax.fori_loop(0, _NCH, body, 0)
        plsc.subcore_barrier()
        for off, cnt in _SLICES:
            pltpu.sync_copy(agg_sh.at[pl.ds(base + off, cnt)],
                            rows_v.at[pl.ds(0, cnt)])
            pltpu.sync_copy(rows_v.at[pl.ds(0, cnt)],
                            out_hbm.at[cid, pl.ds(base + off, cnt)])

    return agg_sc


_agg_sc_hid = _make_agg_sc(_DHID)


def _dinv_block(deg_ref):
    d = deg_ref[...]
    return lax.rsqrt(d[0, :, 0:1] + d[1, :, 0:1] + 1.0)


def _tc1_body(x_ref, w_ref, deg_ref, out_ref):
    h = jnp.dot(x_ref[...], w_ref[...], preferred_element_type=jnp.float32)
    out_ref[...] = h * _dinv_block(deg_ref)


def _tc1(x, w1, deg_parts):
    return pl.pallas_call(
        _tc1_body,
        grid=(_G,),
        in_specs=[
            pl.BlockSpec((_ROWS, _DIN), lambda i: (i, 0)),
            pl.BlockSpec((_DIN, _DHID), lambda i: (0, 0)),
            pl.BlockSpec((_NC, _ROWS, _DHID), lambda i: (0, i, 0)),
        ],
        out_specs=pl.BlockSpec((_ROWS, _DHID), lambda i: (i, 0)),
        out_shape=jax.ShapeDtypeStruct((_N, _DHID), jnp.float32),
    )(x, w1, deg_parts)


def _tc2_body(agg_ref, hs1_ref, deg_ref, b1_ref, w2_ref, out_ref):
    dinv = _dinv_block(deg_ref)
    a = agg_ref[...]
    z = (a[0] + a[1] + hs1_ref[...]) * dinv + b1_ref[0:1, :]
    h1 = jnp.maximum(z, 0.0)
    out_ref[...] = (
        jnp.dot(h1, w2_ref[...], preferred_element_type=jnp.float32) * dinv
    )


def _tc2(agg1, hs1, deg_parts, b1t, w2):
    return pl.pallas_call(
        _tc2_body,
        grid=(_G,),
        in_specs=[
            pl.BlockSpec((_NC, _ROWS, _DHID), lambda i: (0, i, 0)),
            pl.BlockSpec((_ROWS, _DHID), lambda i: (i, 0)),
            pl.BlockSpec((_NC, _ROWS, _DHID), lambda i: (0, i, 0)),
            pl.BlockSpec((8, _DHID), lambda i: (0, 0)),
            pl.BlockSpec((_DHID, _DHID), lambda i: (0, 0)),
        ],
        out_specs=pl.BlockSpec((_ROWS, _DHID), lambda i: (i, 0)),
        out_shape=jax.ShapeDtypeStruct((_N, _DHID), jnp.float32),
    )(agg1, hs1, deg_parts, b1t, w2)


def _tc3_body(agg_ref, hs2_ref, deg_ref, b2_ref, out_ref):
    dinv = _dinv_block(deg_ref)
    a = agg_ref[...]
    z = (a[0, :, :_NCLS] + a[1, :, :_NCLS] + hs2_ref[:, :_NCLS]) * dinv \
        + b2_ref[0:1, :]
    m = jnp.max(z, axis=1, keepdims=True)
    e = jnp.exp(z - m)
    s = jnp.sum(e, axis=1, keepdims=True)
    out_ref[...] = z - m - jnp.log(s)


def _tc3(agg2, hs2, deg_parts, b2t):
    return pl.pallas_call(
        _tc3_body,
        grid=(_G,),
        in_specs=[
            pl.BlockSpec((_NC, _ROWS, _DHID), lambda i: (0, i, 0)),
            pl.BlockSpec((_ROWS, _DHID), lambda i: (i, 0)),
            pl.BlockSpec((_NC, _ROWS, _DHID), lambda i: (0, i, 0)),
            pl.BlockSpec((8, _NCLS), lambda i: (0, 0)),
        ],
        out_specs=pl.BlockSpec((_ROWS, _NCLS), lambda i: (i, 0)),
        out_shape=jax.ShapeDtypeStruct((_N, _NCLS), jnp.float32),
    )(agg2, hs2, deg_parts, b2t)


def kernel(x, edge_index, W1, b1, W2, b2):
    npad = _EPAD - _E
    src = jnp.concatenate(
        [edge_index[0], jnp.zeros((npad,), jnp.int32)]
    ).reshape(_NCH, _W, _B).swapaxes(0, 1)
    dst = jnp.concatenate(
        [edge_index[1],
         _N + (jnp.arange(npad, dtype=jnp.int32) % (_NPAD - _N))]
    ).reshape(_NCH, _W, _B).swapaxes(0, 1)

    zhid = jnp.zeros((_B, _DHID), jnp.float32)
    ohid = jnp.ones((_B, _DHID), jnp.float32)
    b1t = jnp.tile(b1[None, :], (8, 1))
    b2t = jnp.tile(b2[None, :], (8, 1))
    w2p = jnp.pad(W2, ((0, 0), (0, _DHID - _NCLS)))

    deg_parts = _deg_sc(dst, zhid, ohid)
    hs1 = _tc1(x, W1, deg_parts)
    agg1 = _agg_sc_hid(hs1, src, dst, zhid)
    hs2 = _tc2(agg1, hs1, deg_parts, b1t, w2p)
    agg2 = _agg_sc_hid(hs2, src, dst, zhid)
    return _tc3(agg2, hs2, deg_parts, b2t)


# speedup vs baseline: 1.3274x; 1.3274x over previous
import functools

import jax
import jax.numpy as jnp
from jax import lax
from jax.experimental import pallas as pl
from jax.experimental.pallas import tpu as pltpu
from jax.experimental.pallas import tpu_sc as plsc

_N = 10000
_E = 320000
_DIN = 128
_DHID = 128
_NCLS = 64

_NC = 2
_NS = 16
_W = _NC * _NS
_B = 128
_NCH = 79
_EPAD = _W * _B * _NCH
_NPAD = 10112
_RPT = _NPAD // _NS
_SLICES = [(t * _B, min(_B, _RPT - t * _B)) for t in range(-(-_RPT // _B))]

_ROWS = 1000
_G = _N // _ROWS


def _sc_mesh():
    return plsc.VectorSubcoreMesh(core_axis_name="c", subcore_axis_name="s")


@functools.partial(
    pl.kernel,
    mesh=_sc_mesh(),
    out_type=jax.ShapeDtypeStruct((_NC, _NPAD, _DHID), jnp.float32),
    scratch_types=[
        pltpu.VMEM((_NCH, _B), jnp.int32),
        pltpu.VMEM((_B, _DHID), jnp.float32),
        pltpu.VMEM_SHARED((_NPAD, _DHID), jnp.float32),
    ],
)
def _deg_sc(dst_hbm, zeros_hbm, ones_hbm, out_hbm, idx_v, buf_v, deg_sh):
    cid = lax.axis_index("c")
    sid = lax.axis_index("s")
    wid = cid * _NS + sid
    base = sid * _RPT
    pltpu.sync_copy(dst_hbm.at[wid], idx_v)
    pltpu.sync_copy(zeros_hbm, buf_v)
    for off, cnt in _SLICES:
        pltpu.sync_copy(buf_v.at[pl.ds(0, cnt)],
                        deg_sh.at[pl.ds(base + off, cnt)])
    pltpu.sync_copy(ones_hbm, buf_v)
    plsc.subcore_barrier()

    def body(j, carry):
        pltpu.sync_copy(buf_v, deg_sh.at[idx_v.at[j]], add=True)
        return carry

    lax.fori_loop(0, _NCH, body, 0)
    plsc.subcore_barrier()
    for off, cnt in _SLICES:
        pltpu.sync_copy(deg_sh.at[pl.ds(base + off, cnt)],
                        buf_v.at[pl.ds(0, cnt)])
        pltpu.sync_copy(buf_v.at[pl.ds(0, cnt)],
                        out_hbm.at[cid, pl.ds(base + off, cnt)])


def _make_agg_sc(d):
    @functools.partial(
        pl.kernel,
        mesh=_sc_mesh(),
        out_type=jax.ShapeDtypeStruct((_NC, _NPAD, d), jnp.float32),
        scratch_types=[
            pltpu.VMEM((_NCH, _B), jnp.int32),
            pltpu.VMEM((_NCH, _B), jnp.int32),
            pltpu.VMEM((_B, d), jnp.float32),
            pltpu.VMEM_SHARED((_NPAD, d), jnp.float32),
            pltpu.SemaphoreType.DMA,
        ],
    )
    def agg_sc(hs_hbm, src_hbm, dst_hbm, zeros_hbm, out_hbm,
               src_v, dst_v, rows_v, agg_sh, sem):
        cid = lax.axis_index("c")
        sid = lax.axis_index("s")
        wid = cid * _NS + sid
        base = sid * _RPT
        pltpu.sync_copy(src_hbm.at[wid], src_v)
        pltpu.sync_copy(dst_hbm.at[wid], dst_v)
        pltpu.sync_copy(zeros_hbm, rows_v)
        for off, cnt in _SLICES:
            pltpu.sync_copy(rows_v.at[pl.ds(0, cnt)],
                            agg_sh.at[pl.ds(base + off, cnt)])
        plsc.subcore_barrier()

        def body(j, carry):
            pltpu.async_copy(hs_hbm.at[src_v.at[j]], rows_v, sem).wait()
            pltpu.sync_copy(rows_v, agg_sh.at[dst_v.at[j]], add=True)
            return carry

        lax.fori_loop(0, _NCH, body, 0)
        plsc.subcore_barrier()
        for off, cnt in _SLICES:
            pltpu.sync_copy(agg_sh.at[pl.ds(base + off, cnt)],
                            rows_v.at[pl.ds(0, cnt)])
            pltpu.sync_copy(rows_v.at[pl.ds(0, cnt)],
                            out_hbm.at[cid, pl.ds(base + off, cnt)])

    return agg_sc


_agg_sc_hid = _make_agg_sc(_DHID)


def _dinv_block(deg_ref):
    d = deg_ref[...]
    return lax.rsqrt(d[0, :, 0:1] + d[1, :, 0:1] + 1.0)


def _tc1_body(x_ref, w_ref, deg_ref, out_ref):
    h = jnp.dot(x_ref[...], w_ref[...], preferred_element_type=jnp.float32)
    out_ref[...] = h * _dinv_block(deg_ref)


def _tc1(x, w1, deg_parts):
    return pl.pallas_call(
        _tc1_body,
        grid=(_G,),
        in_specs=[
            pl.BlockSpec((_ROWS, _DIN), lambda i: (i, 0)),
            pl.BlockSpec((_DIN, _DHID), lambda i: (0, 0)),
            pl.BlockSpec((_NC, _ROWS, _DHID), lambda i: (0, i, 0)),
        ],
        out_specs=pl.BlockSpec((_ROWS, _DHID), lambda i: (i, 0)),
        out_shape=jax.ShapeDtypeStruct((_N, _DHID), jnp.float32),
    )(x, w1, deg_parts)


def _tc2_body(agg_ref, hs1_ref, deg_ref, b1_ref, w2_ref, out_ref):
    dinv = _dinv_block(deg_ref)
    a = agg_ref[...]
    z = (a[0] + a[1] + hs1_ref[...]) * dinv + b1_ref[0:1, :]
    h1 = jnp.maximum(z, 0.0)
    out_ref[...] = (
        jnp.dot(h1, w2_ref[...], preferred_element_type=jnp.float32) * dinv
    )


def _tc2(agg1, hs1, deg_parts, b1t, w2):
    return pl.pallas_call(
        _tc2_body,
        grid=(_G,),
        in_specs=[
            pl.BlockSpec((_NC, _ROWS, _DHID), lambda i: (0, i, 0)),
            pl.BlockSpec((_ROWS, _DHID), lambda i: (i, 0)),
            pl.BlockSpec((_NC, _ROWS, _DHID), lambda i: (0, i, 0)),
            pl.BlockSpec((8, _DHID), lambda i: (0, 0)),
            pl.BlockSpec((_DHID, _DHID), lambda i: (0, 0)),
        ],
        out_specs=pl.BlockSpec((_ROWS, _DHID), lambda i: (i, 0)),
        out_shape=jax.ShapeDtypeStruct((_N, _DHID), jnp.float32),
    )(agg1, hs1, deg_parts, b1t, w2)


def _tc3_body(agg_ref, hs2_ref, deg_ref, b2_ref, out_ref):
    dinv = _dinv_block(deg_ref)
    a = agg_ref[...]
    z = (a[0, :, :_NCLS] + a[1, :, :_NCLS] + hs2_ref[:, :_NCLS]) * dinv \
        + b2_ref[0:1, :]
    m = jnp.max(z, axis=1, keepdims=True)
    e = jnp.exp(z - m)
    s = jnp.sum(e, axis=1, keepdims=True)
    out_ref[...] = z - m - jnp.log(s)


def _tc3(agg2, hs2, deg_parts, b2t):
    return pl.pallas_call(
        _tc3_body,
        grid=(_G,),
        in_specs=[
            pl.BlockSpec((_NC, _ROWS, _DHID), lambda i: (0, i, 0)),
            pl.BlockSpec((_ROWS, _DHID), lambda i: (i, 0)),
            pl.BlockSpec((_NC, _ROWS, _DHID), lambda i: (0, i, 0)),
            pl.BlockSpec((8, _NCLS), lambda i: (0, 0)),
        ],
        out_specs=pl.BlockSpec((_ROWS, _NCLS), lambda i: (i, 0)),
        out_shape=jax.ShapeDtypeStruct((_N, _NCLS), jnp.float32),
    )(agg2, hs2, deg_parts, b2t)


def kernel(x, edge_index, W1, b1, W2, b2):
    npad = _EPAD - _E
    src = jnp.concatenate(
        [edge_index[0], jnp.zeros((npad,), jnp.int32)]).reshape(_W, _NCH, _B)
    dst = jnp.concatenate(
        [edge_index[1],
         _N + (jnp.arange(npad, dtype=jnp.int32) % 16)]).reshape(_W, _NCH, _B)

    zhid = jnp.zeros((_B, _DHID), jnp.float32)
    ohid = jnp.ones((_B, _DHID), jnp.float32)
    b1t = jnp.tile(b1[None, :], (8, 1))
    b2t = jnp.tile(b2[None, :], (8, 1))
    w2p = jnp.pad(W2, ((0, 0), (0, _DHID - _NCLS)))

    deg_parts = _deg_sc(dst, zhid, ohid)
    hs1 = _tc1(x, W1, deg_parts)
    agg1 = _agg_sc_hid(hs1, src, dst, zhid)
    hs2 = _tc2(agg1, hs1, deg_parts, b1t, w2p)
    agg2 = _agg_sc_hid(hs2, src, dst, zhid)
    return _tc3(agg2, hs2, deg_parts, b2t)
